# Initial kernel scaffold; baseline (speedup 1.0000x reference)
import jax
import jax.numpy as jnp
from jax.experimental import pallas as pl


def kernel(g1_x, g1_edge_index, g1_batch, g2_x, g2_edge_index, g2_batch, d1, d2, nn1_w1, nn1_b1, nn1_w2, nn1_b2, ln1_w, ln1_b, nn2_w1, nn2_b1, nn2_w2, nn2_b2, ln2_w, ln2_b, fc1_w, fc1_b, fc2_w, fc2_b, out_w, out_b):
    raise NotImplementedError("write your pallas kernel here")


# speedup vs baseline: 4.6168x; 4.6168x over previous
import functools

import jax
import jax.numpy as jnp
from jax import lax
from jax.experimental import pallas as pl
from jax.experimental.pallas import tpu as pltpu
from jax.experimental.pallas import tpu_sc as plsc

N = 10000
CIN = 128
C = 64
B = 64
B2 = 2 * B
E = 320000
EPS = 1e-5

M = 2 * N
RB = 1000
NBLK = M // RB

NSUB = 16
CW = 128
CHUNKS = 160
E_PAD = NSUB * CHUNKS * CW
ACC_ROWS = 10240
SUB_ROWS = ACC_ROWS // NSUB
OUT_ROWS = N // NSUB
DUMMY_ROW = N + 8

_HIGH = lax.Precision.HIGHEST


def _dot(a, b):
    return lax.dot_general(a, b, (((1,), (0,)), ((), ())),
                           precision=_HIGH, preferred_element_type=jnp.float32)


def _bdot(a, b):
    return lax.dot_general(a.astype(jnp.bfloat16), b.astype(jnp.bfloat16),
                           (((1,), (0,)), ((), ())),
                           preferred_element_type=jnp.float32)


def _sc_phase(y_hbm, out_hbm, src_v, dst_v, buf0, buf1, acc, sem0, sem1,
              z_hbm, c, s):
    pltpu.sync_copy(z_hbm, acc.at[pl.ds(s * SUB_ROWS, SUB_ROWS)])
    plsc.subcore_barrier()

    pltpu.async_copy(y_hbm.at[src_v.at[0]], buf0, sem0)

    @pl.loop(0, CHUNKS - 2, step=2)
    def _(j):
        pltpu.async_copy(y_hbm.at[src_v.at[j + 1]], buf1, sem1)
        pltpu.make_async_copy(y_hbm.at[src_v.at[j]], buf0, sem0).wait()
        pltpu.sync_copy(buf0, acc.at[dst_v.at[j]], add=True)
        pltpu.async_copy(y_hbm.at[src_v.at[j + 2]], buf0, sem0)
        pltpu.make_async_copy(y_hbm.at[src_v.at[j + 1]], buf1, sem1).wait()
        pltpu.sync_copy(buf1, acc.at[dst_v.at[j + 1]], add=True)

    pltpu.async_copy(y_hbm.at[src_v.at[CHUNKS - 1]], buf1, sem1)
    pltpu.make_async_copy(y_hbm.at[src_v.at[CHUNKS - 2]], buf0, sem0).wait()
    pltpu.sync_copy(buf0, acc.at[dst_v.at[CHUNKS - 2]], add=True)
    pltpu.make_async_copy(y_hbm.at[src_v.at[CHUNKS - 1]], buf1, sem1).wait()
    pltpu.sync_copy(buf1, acc.at[dst_v.at[CHUNKS - 1]], add=True)

    plsc.subcore_barrier()
    pltpu.sync_copy(
        acc.at[pl.ds(s * OUT_ROWS, OUT_ROWS)],
        out_hbm.at[pl.ds(c * N + s * OUT_ROWS, OUT_ROWS)])


_SC_SCRATCH = [
    pltpu.VMEM((CHUNKS, CW), jnp.int32),
    pltpu.VMEM((CHUNKS, CW), jnp.int32),
    pltpu.VMEM((CW, C), jnp.float32),
    pltpu.VMEM((CW, C), jnp.float32),
    pltpu.VMEM_SHARED((ACC_ROWS, C), jnp.float32),
    pltpu.SemaphoreType.DMA,
    pltpu.SemaphoreType.DMA,
]
def _sc_mesh():
    return plsc.VectorSubcoreMesh(core_axis_name="c", subcore_axis_name="s")


def _sc_segsum(y, src_idx, dst_idx, zrows):
    @functools.partial(
        pl.kernel,
        out_type=jax.ShapeDtypeStruct((M, C), jnp.float32),
        mesh=_sc_mesh(),
        compiler_params=pltpu.CompilerParams(use_tc_tiling_on_sc=False),
        scratch_types=_SC_SCRATCH,
    )
    def k(y_hbm, src_hbm, dst_hbm, z_hbm, out_hbm,
          src_v, dst_v, buf0, buf1, acc, sem0, sem1):
        c = lax.axis_index("c")
        s = lax.axis_index("s")
        w = c * NSUB + s
        pltpu.sync_copy(src_hbm.at[w], src_v)
        pltpu.sync_copy(dst_hbm.at[w], dst_v)
        _sc_phase(y_hbm, out_hbm, src_v, dst_v, buf0, buf1, acc, sem0, sem1,
                  z_hbm, c, s)

    return k(y, src_idx, dst_idx, zrows)


def _sc_segsum2(ya, yb, src_idx, dst_idx, zrows):
    out_t = jax.ShapeDtypeStruct((M, C), jnp.float32)

    @functools.partial(
        pl.kernel,
        out_type=[out_t, out_t],
        mesh=_sc_mesh(),
        compiler_params=pltpu.CompilerParams(use_tc_tiling_on_sc=False),
        scratch_types=_SC_SCRATCH,
    )
    def k(ya_hbm, yb_hbm, src_hbm, dst_hbm, z_hbm, oa_hbm, ob_hbm,
          src_v, dst_v, buf0, buf1, acc, sem0, sem1):
        c = lax.axis_index("c")
        s = lax.axis_index("s")
        w = c * NSUB + s
        pltpu.sync_copy(src_hbm.at[w], src_v)
        pltpu.sync_copy(dst_hbm.at[w], dst_v)
        _sc_phase(ya_hbm, oa_hbm, src_v, dst_v, buf0, buf1, acc, sem0, sem1,
                  z_hbm, c, s)
        plsc.subcore_barrier()
        _sc_phase(yb_hbm, ob_hbm, src_v, dst_v, buf0, buf1, acc, sem0, sem1,
                  z_hbm, c, s)

    return k(ya, yb, src_idx, dst_idx, zrows)


_ROWS = pl.BlockSpec((RB, C), lambda i: (i, 0))
_BR = pl.BlockSpec((1, 1, RB), lambda i: (i, 0, 0))
_BC = pl.BlockSpec((1, RB, 1), lambda i: (i, 0, 0))
_STAT = pl.BlockSpec((B2, C), lambda i: (0, 0))
_DEG = pl.BlockSpec((1, B2), lambda i: (0, 0))


def _wspec(*shape):
    return pl.BlockSpec(shape, lambda i: tuple(0 for _ in shape))


def _tc_mlp_stats(x, agg, w1, b1, w2, b2, br):
    cin = x.shape[1]

    def body(x_ref, agg_ref, w1_ref, b1_ref, w2_ref, b2_ref, br_ref,
             h_ref, gsum_ref, gsq_ref, deg_ref):
        i = pl.program_id(0)
        s = x_ref[...] + agg_ref[...]
        t = jnp.maximum(_bdot(s, w1_ref[...]) + b1_ref[...], 0.0)
        h = _bdot(t, w2_ref[...]) + b2_ref[...]
        h_ref[...] = h
        oh = (lax.broadcasted_iota(jnp.int32, (B2, RB), 0)
              == br_ref[0]).astype(jnp.float32)

        @pl.when(i == 0)
        def _():
            gsum_ref[...] = jnp.zeros_like(gsum_ref)
            gsq_ref[...] = jnp.zeros_like(gsq_ref)
            deg_ref[...] = jnp.zeros_like(deg_ref)

        gsum_ref[...] += _dot(oh, h)
        gsq_ref[...] += _dot(oh, h * h)
        deg_ref[...] += jnp.sum(oh, axis=1)[None, :]

    return pl.pallas_call(
        body,
        grid=(NBLK,),
        in_specs=[pl.BlockSpec((RB, cin), lambda i: (i, 0)),
                  pl.BlockSpec((RB, cin), lambda i: (i, 0)),
                  _wspec(cin, C), _wspec(1, C), _wspec(C, C), _wspec(1, C),
                  _BR],
        out_specs=[_ROWS, _STAT, _STAT, _DEG],
        out_shape=[jax.ShapeDtypeStruct((M, C), jnp.float32),
                   jax.ShapeDtypeStruct((B2, C), jnp.float32),
                   jax.ShapeDtypeStruct((B2, C), jnp.float32),
                   jax.ShapeDtypeStruct((1, B2), jnp.float32)],
    )(x, agg, w1, b1, w2, b2, br)


def _ln_consts(gsum_ref, gsq_ref, deg_ref):
    norm = jnp.maximum(deg_ref[0], 1.0) * float(C)
    mean = jnp.sum(gsum_ref[...], axis=1) / norm
    ex2 = jnp.sum(gsq_ref[...], axis=1) / norm
    var = ex2 - mean * mean
    v = var + EPS
    inv = lax.rsqrt(v)
    inv = inv * (1.5 - 0.5 * v * inv * inv)
    mean_rep = jnp.broadcast_to(mean[:, None], (B2, C))
    inv_rep = jnp.broadcast_to(inv[:, None], (B2, C))
    return mean_rep, inv_rep


def _ln_block(h_ref, gsum_ref, gsq_ref, deg_ref, lnw_ref, lnb_ref, bc_ref):
    mean_rep, inv_rep = _ln_consts(gsum_ref, gsq_ref, deg_ref)
    oh_t = (lax.broadcasted_iota(jnp.int32, (RB, B2), 1)
            == bc_ref[0]).astype(jnp.float32)
    mean_n = _dot(oh_t, mean_rep)
    inv_n = _dot(oh_t, inv_rep)
    return jnp.maximum((h_ref[...] - mean_n) * inv_n * lnw_ref[...]
                       + lnb_ref[...], 0.0)


def _tc_ln(h, gsum, gsq, deg, lnw, lnb, bc):
    def body(h_ref, gsum_ref, gsq_ref, deg_ref, lnw_ref, lnb_ref, bc_ref,
             o_ref):
        o_ref[...] = _ln_block(h_ref, gsum_ref, gsq_ref, deg_ref,
                               lnw_ref, lnb_ref, bc_ref)

    return pl.pallas_call(
        body,
        grid=(NBLK,),
        in_specs=[_ROWS, _STAT, _STAT, _DEG, _wspec(1, C), _wspec(1, C), _BC],
        out_specs=_ROWS,
        out_shape=jax.ShapeDtypeStruct((M, C), jnp.float32),
    )(h, gsum, gsq, deg, lnw, lnb, bc)


def _tc_ln_pool(h, gsum, gsq, deg, lnw, lnb, bc, br):
    def body(h_ref, gsum_ref, gsq_ref, deg_ref, lnw_ref, lnb_ref,
             bc_ref, br_ref, pool_ref):
        i = pl.program_id(0)
        hn = _ln_block(h_ref, gsum_ref, gsq_ref, deg_ref,
                       lnw_ref, lnb_ref, bc_ref)
        oh = (lax.broadcasted_iota(jnp.int32, (B2, RB), 0)
              == br_ref[0]).astype(jnp.float32)

        @pl.when(i == 0)
        def _():
            pool_ref[...] = jnp.zeros_like(pool_ref)

        pool_ref[...] += _dot(oh, hn)

    return pl.pallas_call(
        body,
        grid=(NBLK,),
        in_specs=[_ROWS, _STAT, _STAT, _DEG, _wspec(1, C), _wspec(1, C),
                  _BC, _BR],
        out_specs=_STAT,
        out_shape=jax.ShapeDtypeStruct((B2, C), jnp.float32),
    )(h, gsum, gsq, deg, lnw, lnb, bc, br)


def _tc_head(pool, deg, d1, d2, fc1w, fc1b, fc2w, fc2b, ow, ob):
    def body(pool_ref, deg_ref, d1_ref, d2_ref, f1w_ref, f1b_ref,
             f2w_ref, f2b_ref, ow_ref, ob_ref, o_ref):
        cnt = jnp.maximum(deg_ref[0], 1.0)
        pool = pool_ref[...]
        emb = pool + pool / cnt[:, None]
        comb = jnp.concatenate(
            [emb[:B, :], emb[B:, :], d1_ref[...], d2_ref[...],
             jnp.zeros((B, 6), jnp.float32)], axis=1)
        hh = jnp.maximum(_bdot(comb, f1w_ref[...]) + f1b_ref[...], 0.0)
        hh = jnp.maximum(_bdot(hh, f2w_ref[...]) + f2b_ref[...], 0.0)
        o_ref[...] = _bdot(hh, ow_ref[...]) + ob_ref[...]

    return pl.pallas_call(
        body,
        out_shape=jax.ShapeDtypeStruct((B, 1), jnp.float32),
    )(pool, deg, d1, d2, fc1w, fc1b, fc2w, fc2b, ow, ob)


def kernel(g1_x, g1_edge_index, g1_batch, g2_x, g2_edge_index, g2_batch,
           d1, d2, nn1_w1, nn1_b1, nn1_w2, nn1_b2, ln1_w, ln1_b,
           nn2_w1, nn2_b1, nn2_w2, nn2_b2, ln2_w, ln2_b,
           fc1_w, fc1_b, fc2_w, fc2_b, out_w, out_b):
    x_all = jnp.concatenate([g1_x, g2_x], axis=0)
    batch_all = jnp.concatenate([g1_batch, g2_batch + B]).astype(jnp.int32)
    br = batch_all.reshape(NBLK, 1, RB)
    bc = batch_all.reshape(NBLK, RB, 1)

    npad = E_PAD - E
    src = jnp.stack([
        jnp.concatenate([g1_edge_index[0], jnp.zeros((npad,), jnp.int32)]),
        jnp.concatenate([g2_edge_index[0] + N, jnp.full((npad,), N, jnp.int32)]),
    ]).reshape(2 * NSUB, CHUNKS, CW)
    pad_dst = jnp.full((npad,), DUMMY_ROW, jnp.int32)
    dst = jnp.stack([
        jnp.concatenate([g1_edge_index[1], pad_dst]),
        jnp.concatenate([g2_edge_index[1], pad_dst]),
    ]).reshape(2 * NSUB, CHUNKS, CW)
    zrows = jnp.zeros((SUB_ROWS, C), jnp.float32)

    fc1w_p = jnp.concatenate([fc1_w, jnp.zeros((6, fc1_w.shape[1]),
                                               jnp.float32)], axis=0)

    aL, aR = _sc_segsum2(x_all[:, :C], x_all[:, C:], src, dst, zrows)
    agg1 = jnp.concatenate([aL, aR], axis=1)
    h1, gsum1, gsq1, deg = _tc_mlp_stats(x_all, agg1, nn1_w1, nn1_b1[None],
                                         nn1_w2, nn1_b2[None], br)
    h1r = _tc_ln(h1, gsum1, gsq1, deg, ln1_w[None], ln1_b[None], bc)
    agg2 = _sc_segsum(h1r, src, dst, zrows)
    h2, gsum2, gsq2, _ = _tc_mlp_stats(h1r, agg2, nn2_w1, nn2_b1[None],
                                       nn2_w2, nn2_b2[None], br)
    pool = _tc_ln_pool(h2, gsum2, gsq2, deg, ln2_w[None], ln2_b[None], bc, br)
    return _tc_head(pool, deg, d1, d2, fc1w_p, fc1_b[None], fc2_w,
                    fc2_b[None], out_w, out_b[None])
